# baseline (speedup 1.0000x reference)
import functools

import jax
import jax.numpy as jnp
from jax import lax
from jax.experimental import pallas as pl
from jax.experimental.pallas import tpu as pltpu
from jax.experimental.pallas import tpu_sc as plsc

N = 10000
NP = 10240
E = 320000
DF = 128
DH = 64
DO = 5
WPAD = 16
SCALE = 512.0

NC = 2
NS = 16
NW = NC * NS
CH = 125
NCHUNK = 80
RING = 4
RPT = NP // NS

_MESH = plsc.VectorSubcoreMesh(
    core_axis_name="c", subcore_axis_name="s", num_cores=NC, num_subcores=NS)
_SC_PARAMS = pltpu.CompilerParams(use_tc_tiling_on_sc=False)


def _sc_edge_agg(width, dtype):
  @functools.partial(
      pl.kernel,
      out_type=jax.ShapeDtypeStruct((NC, NP, width), dtype),
      mesh=_MESH,
      compiler_params=_SC_PARAMS,
      scratch_types=[
          pltpu.VMEM_SHARED((NP, width), dtype),
          pltpu.VMEM((NCHUNK, CH), jnp.int32),
          pltpu.VMEM((NCHUNK, CH), jnp.int32),
      ] + [pltpu.VMEM((CH, width), dtype)] * RING
        + [pltpu.SemaphoreType.DMA] * RING,
  )
  def body(table_hbm, edge_hbm, out_hbm, acc, srcs, dsts, *bufs):
    rows = bufs[:RING]
    gsem = bufs[RING:]
    c = lax.axis_index("c")
    s = lax.axis_index("s")
    wid = c * NS + s
    pltpu.sync_copy(table_hbm.at[pl.ds(s * RPT, RPT)],
                    acc.at[pl.ds(s * RPT, RPT)])
    pltpu.sync_copy(edge_hbm.at[0, pl.ds(wid * NCHUNK, NCHUNK)], srcs)
    pltpu.sync_copy(edge_hbm.at[1, pl.ds(wid * NCHUNK, NCHUNK)], dsts)
    plsc.subcore_barrier()

    for b in range(RING):
      pltpu.async_copy(table_hbm.at[srcs.at[b]], rows[b], gsem[b])

    def outer(i, carry):
      for b in range(RING):
        j = RING * i + b
        pltpu.make_async_copy(table_hbm.at[srcs.at[j]], rows[b],
                              gsem[b]).wait()
        pltpu.sync_copy(rows[b], acc.at[dsts.at[j]], add=True)

        @pl.when(j + RING < NCHUNK)
        def _issue():
          pltpu.async_copy(table_hbm.at[srcs.at[j + RING]], rows[b], gsem[b])
      return carry

    lax.fori_loop(0, NCHUNK // RING, outer, None)
    plsc.subcore_barrier()
    pltpu.sync_copy(acc.at[pl.ds(s * RPT, RPT)],
                    out_hbm.at[c, pl.ds(s * RPT, RPT)])

  return body


@functools.partial(
    pl.kernel,
    out_type=[jax.ShapeDtypeStruct((NC, NP, WPAD), jnp.float32),
              jax.ShapeDtypeStruct((2, NW * NCHUNK, CH), jnp.int32)],
    mesh=_MESH,
    compiler_params=_SC_PARAMS,
    scratch_types=[
        pltpu.VMEM_SHARED((NP, WPAD), jnp.float32),
        pltpu.VMEM((NCHUNK, CH), jnp.int32),
        pltpu.VMEM((NCHUNK, CH), jnp.int32),
        pltpu.SemaphoreType.DMA,
        pltpu.VMEM((CH, WPAD), jnp.float32),
        pltpu.SemaphoreType.DMA,
    ],
)
def _sc_degree(ones_hbm, edge_hbm, out_hbm, ei_out, acc, dsts, srcs, ssem,
               rows, sem):
  c = lax.axis_index("c")
  s = lax.axis_index("s")
  wid = c * NS + s
  pltpu.sync_copy(ones_hbm.at[pl.ds(s * RPT, RPT)],
                  acc.at[pl.ds(s * RPT, RPT)])
  pltpu.sync_copy(edge_hbm.at[1, pl.ds(wid * NCHUNK, NCHUNK)], dsts)
  pltpu.sync_copy(ones_hbm.at[pl.ds(0, CH)], rows)
  srccp = pltpu.async_copy(edge_hbm.at[0, pl.ds(wid * NCHUNK, NCHUNK)],
                           srcs, ssem)
  pltpu.sync_copy(dsts, ei_out.at[1, pl.ds(wid * NCHUNK, NCHUNK)])
  srccp.wait()
  pltpu.sync_copy(srcs, ei_out.at[0, pl.ds(wid * NCHUNK, NCHUNK)])
  plsc.subcore_barrier()

  GRP = 8

  def outer(i, carry):
    for b in range(GRP):
      pltpu.async_copy(rows, acc.at[dsts.at[GRP * i + b]], sem, add=True)
    for b in range(GRP):
      pltpu.make_async_copy(rows, acc.at[dsts.at[GRP * i + b]], sem).wait()
    return carry

  lax.fori_loop(0, NCHUNK // GRP, outer, None)
  plsc.subcore_barrier()
  pltpu.sync_copy(acc.at[pl.ds(s * RPT, RPT)],
                  out_hbm.at[c, pl.ds(s * RPT, RPT)])


def _tc0_body(x_ref, w1_ref, h_ref):
  h_ref[...] = jnp.dot(x_ref[...], w1_ref[...],
                       preferred_element_type=jnp.float32)


def _tc1_body(h_ref, degp_ref, hsq_ref, dinv_ref):
  deg = degp_ref[0, :, :1] + degp_ref[1, :, :1] - 1.0
  dinv = lax.rsqrt(deg)
  hsq_ref[...] = jnp.rint(h_ref[...] * (dinv * SCALE)).astype(jnp.int16)
  dinv_ref[...] = dinv


def _tc2_body(p_ref, hsq_ref, dinv_ref, b1_ref, w2p_ref, zs_ref):
  dinv = dinv_ref[...]
  acc = (p_ref[0].astype(jnp.float32) + p_ref[1].astype(jnp.float32)
         - hsq_ref[...].astype(jnp.float32))
  a = dinv * acc * (1.0 / SCALE) + b1_ref[...]
  r = jnp.maximum(a, 0.0)
  z = jnp.dot(r, w2p_ref[...], preferred_element_type=jnp.float32)
  zs_ref[...] = z * dinv


def _tc3_body(q_ref, zs_ref, dinv_ref, b2p_ref, o_ref):
  a = dinv_ref[...] * (q_ref[0] + q_ref[1] - zs_ref[...]) + b2p_ref[...]
  col = lax.broadcasted_iota(jnp.int32, a.shape, 1)
  l = jnp.where(col < DO, a, -jnp.inf)
  m = jnp.max(l, axis=1, keepdims=True)
  ssum = jnp.sum(jnp.where(col < DO, jnp.exp(l - m), 0.0),
                 axis=1, keepdims=True)
  o_ref[...] = (l - m - jnp.log(ssum))[:N, :DO]


_agg64 = _sc_edge_agg(DH, jnp.int16)
_agg16 = _sc_edge_agg(WPAD, jnp.float32)

_tc0 = pl.pallas_call(
    _tc0_body,
    out_shape=jax.ShapeDtypeStruct((NP, DH), jnp.float32))
_tc1 = pl.pallas_call(
    _tc1_body,
    out_shape=[jax.ShapeDtypeStruct((NP, DH), jnp.int16),
               jax.ShapeDtypeStruct((NP, 1), jnp.float32)])
_tc2 = pl.pallas_call(
    _tc2_body,
    out_shape=jax.ShapeDtypeStruct((NP, WPAD), jnp.float32))
_tc3 = pl.pallas_call(
    _tc3_body,
    out_shape=jax.ShapeDtypeStruct((N, DO), jnp.float32))


@jax.jit
def kernel(x, edge_index, W1, b1, W2, b2):
  ei = edge_index.astype(jnp.int32).reshape(2, NW * NCHUNK, CH)
  xp = jnp.pad(x, ((0, NP - N), (0, 0)))
  ones16 = jnp.ones((NP, WPAD), jnp.float32)
  W2p = jnp.pad(W2, ((0, 0), (0, WPAD - DO)))
  b1r = b1.reshape(1, DH)
  b2p = jnp.pad(b2, (0, WPAD - DO)).reshape(1, WPAD)

  h = _tc0(xp, W1)
  degp, ei2 = _sc_degree(ones16, ei)
  hsq, dinv = _tc1(h, degp)
  p = _agg64(hsq, ei2)
  zs = _tc2(p, hsq, dinv, b1r, W2p)
  q = _agg16(zs, ei2)
  return _tc3(q, zs, dinv, b2p)


# speedup vs baseline: 1.0238x; 1.0238x over previous
import functools

import jax
import jax.numpy as jnp
from jax import lax
from jax.experimental import pallas as pl
from jax.experimental.pallas import tpu as pltpu
from jax.experimental.pallas import tpu_sc as plsc

N = 10000
NP = 10240
E = 320000
DF = 128
DH = 64
DO = 5
WPAD = 16
SCALE = 512.0

NC = 2
NS = 16
NW = NC * NS
CH = 125
NCHUNK = 80
RING = 4
NBUF = 2 * RING
RPT = NP // NS

_MESH = plsc.VectorSubcoreMesh(
    core_axis_name="c", subcore_axis_name="s", num_cores=NC, num_subcores=NS)
_SC_PARAMS = pltpu.CompilerParams(use_tc_tiling_on_sc=False)


def _sc_edge_agg(width, dtype):
  @functools.partial(
      pl.kernel,
      out_type=jax.ShapeDtypeStruct((NC, NP, width), dtype),
      mesh=_MESH,
      compiler_params=_SC_PARAMS,
      scratch_types=[
          pltpu.VMEM_SHARED((NP, width), dtype),
          pltpu.VMEM((NCHUNK, CH), jnp.int32),
          pltpu.VMEM((NCHUNK, CH), jnp.int32),
      ] + [pltpu.VMEM((CH, width), dtype)] * NBUF
        + [pltpu.SemaphoreType.DMA] * NBUF
        + [pltpu.SemaphoreType.DMA] * NBUF,
  )
  def body(table_hbm, edge_hbm, out_hbm, acc, srcs, dsts, *bufs):
    rows = bufs[:NBUF]
    gsem = bufs[NBUF:2 * NBUF]
    ssem = bufs[2 * NBUF:]
    c = lax.axis_index("c")
    s = lax.axis_index("s")
    wid = c * NS + s
    pltpu.sync_copy(table_hbm.at[pl.ds(s * RPT, RPT)],
                    acc.at[pl.ds(s * RPT, RPT)])
    pltpu.sync_copy(edge_hbm.at[0, pl.ds(wid * NCHUNK, NCHUNK)], srcs)
    pltpu.sync_copy(edge_hbm.at[1, pl.ds(wid * NCHUNK, NCHUNK)], dsts)
    plsc.subcore_barrier()

    for b in range(RING):
      pltpu.async_copy(table_hbm.at[srcs.at[b]], rows[b], gsem[b])

    def outer(i, carry):
      for k in range(NBUF):
        j = NBUF * i + k
        b = k
        bg = (k + RING) % NBUF
        pltpu.make_async_copy(table_hbm.at[srcs.at[j]], rows[b],
                              gsem[b]).wait()
        pltpu.async_copy(rows[b], acc.at[dsts.at[j]], ssem[b], add=True)

        @pl.when(j >= RING)
        def _drain():
          pltpu.make_async_copy(rows[bg], acc.at[dsts.at[j - RING]],
                                ssem[bg]).wait()

        @pl.when(j + RING < NCHUNK)
        def _issue():
          pltpu.async_copy(table_hbm.at[srcs.at[j + RING]], rows[bg],
                           gsem[bg])
      return carry

    lax.fori_loop(0, NCHUNK // NBUF, outer, None)
    for k in range(NBUF - RING, NBUF):
      j = NCHUNK - NBUF + k
      pltpu.make_async_copy(rows[k], acc.at[dsts.at[j]], ssem[k]).wait()
    plsc.subcore_barrier()
    pltpu.sync_copy(acc.at[pl.ds(s * RPT, RPT)],
                    out_hbm.at[c, pl.ds(s * RPT, RPT)])

  return body


@functools.partial(
    pl.kernel,
    out_type=jax.ShapeDtypeStruct((NC, NP, WPAD), jnp.float32),
    mesh=_MESH,
    compiler_params=_SC_PARAMS,
    scratch_types=[
        pltpu.VMEM_SHARED((NP, WPAD), jnp.float32),
        pltpu.VMEM((NCHUNK, CH), jnp.int32),
        pltpu.VMEM((CH, WPAD), jnp.float32),
        pltpu.SemaphoreType.DMA,
    ],
)
def _sc_degree(ones_hbm, edge_hbm, out_hbm, acc, dsts, rows, sem):
  c = lax.axis_index("c")
  s = lax.axis_index("s")
  wid = c * NS + s
  pltpu.sync_copy(ones_hbm.at[pl.ds(s * RPT, RPT)],
                  acc.at[pl.ds(s * RPT, RPT)])
  pltpu.sync_copy(edge_hbm.at[1, pl.ds(wid * NCHUNK, NCHUNK)], dsts)
  pltpu.sync_copy(ones_hbm.at[pl.ds(0, CH)], rows)
  plsc.subcore_barrier()

  GRP = 8

  def outer(i, carry):
    for b in range(GRP):
      pltpu.async_copy(rows, acc.at[dsts.at[GRP * i + b]], sem, add=True)
    for b in range(GRP):
      pltpu.make_async_copy(rows, acc.at[dsts.at[GRP * i + b]], sem).wait()
    return carry

  lax.fori_loop(0, NCHUNK // GRP, outer, None)
  plsc.subcore_barrier()
  pltpu.sync_copy(acc.at[pl.ds(s * RPT, RPT)],
                  out_hbm.at[c, pl.ds(s * RPT, RPT)])


def _tc0_body(x_ref, w1_ref, h_ref):
  h_ref[...] = jnp.dot(x_ref[...], w1_ref[...],
                       preferred_element_type=jnp.float32)


def _tc1_body(h_ref, degp_ref, hsq_ref, dinv_ref):
  deg = degp_ref[0, :, :1] + degp_ref[1, :, :1] - 1.0
  dinv = lax.rsqrt(deg)
  hsq_ref[...] = jnp.rint(h_ref[...] * (dinv * SCALE)).astype(jnp.int16)
  dinv_ref[...] = dinv


def _tc2_body(p_ref, hsq_ref, dinv_ref, b1_ref, w2p_ref, zs_ref):
  dinv = dinv_ref[...]
  acc = (p_ref[0].astype(jnp.float32) + p_ref[1].astype(jnp.float32)
         - hsq_ref[...].astype(jnp.float32))
  a = dinv * acc * (1.0 / SCALE) + b1_ref[...]
  r = jnp.maximum(a, 0.0)
  z = jnp.dot(r, w2p_ref[...], preferred_element_type=jnp.float32)
  zs_ref[...] = z * dinv


def _tc3_body(q_ref, zs_ref, dinv_ref, b2p_ref, o_ref):
  a = dinv_ref[...] * (q_ref[0] + q_ref[1] - zs_ref[...]) + b2p_ref[...]
  col = lax.broadcasted_iota(jnp.int32, a.shape, 1)
  l = jnp.where(col < DO, a, -jnp.inf)
  m = jnp.max(l, axis=1, keepdims=True)
  ssum = jnp.sum(jnp.where(col < DO, jnp.exp(l - m), 0.0),
                 axis=1, keepdims=True)
  o_ref[...] = (l - m - jnp.log(ssum))[:N, :DO]


_agg64 = _sc_edge_agg(DH, jnp.int16)
_agg16 = _sc_edge_agg(WPAD, jnp.float32)

_tc0 = pl.pallas_call(
    _tc0_body,
    out_shape=jax.ShapeDtypeStruct((NP, DH), jnp.float32))
_tc1 = pl.pallas_call(
    _tc1_body,
    out_shape=[jax.ShapeDtypeStruct((NP, DH), jnp.int16),
               jax.ShapeDtypeStruct((NP, 1), jnp.float32)])
_tc2 = pl.pallas_call(
    _tc2_body,
    out_shape=jax.ShapeDtypeStruct((NP, WPAD), jnp.float32))
_tc3 = pl.pallas_call(
    _tc3_body,
    out_shape=jax.ShapeDtypeStruct((N, DO), jnp.float32))


@jax.jit
def kernel(x, edge_index, W1, b1, W2, b2):
  ei = edge_index.astype(jnp.int32).reshape(2, NW * NCHUNK, CH)
  xp = jnp.pad(x, ((0, NP - N), (0, 0)))
  ones16 = jnp.ones((NP, WPAD), jnp.float32)
  W2p = jnp.pad(W2, ((0, 0), (0, WPAD - DO)))
  b1r = b1.reshape(1, DH)
  b2p = jnp.pad(b2, (0, WPAD - DO)).reshape(1, WPAD)

  h = _tc0(xp, W1)
  degp = _sc_degree(ones16, ei)
  hsq, dinv = _tc1(h, degp)
  p = _agg64(hsq, ei)
  zs = _tc2(p, hsq, dinv, b1r, W2p)
  q = _agg16(zs, ei)
  return _tc3(q, zs, dinv, b2p)
